# baseline (speedup 1.0000x reference)
import functools

import jax
import jax.numpy as jnp
from jax import lax
from jax.experimental import pallas as pl
from jax.experimental.pallas import tpu as pltpu
from jax.experimental.pallas import tpu_sc as plsc

BATCH = 4096
SEQ = 200
EMBED = 64
N_ROWS = BATCH * SEQ
NUM_WORKERS = 32
ROWS_PER_WORKER = N_ROWS // NUM_WORKERS
CHUNK = 128
NUM_CHUNKS = ROWS_PER_WORKER // CHUNK

_MESH = plsc.VectorSubcoreMesh(
    core_axis_name="c", subcore_axis_name="s", num_cores=2, num_subcores=16
)


@functools.partial(
    pl.kernel,
    out_type=jax.ShapeDtypeStruct((N_ROWS, EMBED), jnp.float32),
    mesh=_MESH,
    compiler_params=pltpu.CompilerParams(use_tc_tiling_on_sc=False),
    scratch_types=[
        pltpu.VMEM((NUM_CHUNKS, CHUNK), jnp.int32),
        pltpu.VMEM((NUM_CHUNKS, CHUNK), jnp.int32),
        pltpu.VMEM((CHUNK, EMBED), jnp.float32),
        pltpu.VMEM((CHUNK, EMBED), jnp.float32),
        pltpu.SemaphoreType.DMA,
        pltpu.SemaphoreType.DMA,
    ],
)
def _embed_sum(notes_hbm, times_hbm, tok_hbm, pos_hbm, out_hbm,
               idx_n, idx_t, buf_n, buf_t, sem_n, sem_t):
    w = lax.axis_index("s") * 2 + lax.axis_index("c")
    pltpu.sync_copy(notes_hbm.at[w], idx_n)
    pltpu.sync_copy(times_hbm.at[w], idx_t)

    def chunk_body(g, carry):
        cp_n = pltpu.async_copy(tok_hbm.at[idx_n.at[g]], buf_n, sem_n)
        cp_n.wait()
        cp_t = pltpu.async_copy(pos_hbm.at[idx_t.at[g]], buf_n, sem_t, add=True)
        cp_t.wait()
        pltpu.sync_copy(buf_n, out_hbm.at[pl.ds(w * ROWS_PER_WORKER + g * CHUNK, CHUNK)])
        return carry

    lax.fori_loop(0, NUM_CHUNKS, chunk_body, 0)


def kernel(x, token_emb, pos_emb):
    notes = x[:, 0, :].astype(jnp.int32).reshape(NUM_WORKERS, NUM_CHUNKS, CHUNK)
    times = x[:, 1, :].astype(jnp.int32).reshape(NUM_WORKERS, NUM_CHUNKS, CHUNK)
    out = _embed_sum(notes, times, token_emb, pos_emb)
    return out.reshape(BATCH, SEQ, EMBED)


# speedup vs baseline: 1.3253x; 1.3253x over previous
import functools

import jax
import jax.numpy as jnp
from jax import lax
from jax.experimental import pallas as pl
from jax.experimental.pallas import tpu as pltpu
from jax.experimental.pallas import tpu_sc as plsc

BATCH = 4096
SEQ = 200
EMBED = 64
N_ROWS = BATCH * SEQ
NUM_WORKERS = 32
ROWS_PER_WORKER = N_ROWS // NUM_WORKERS
CHUNK = 128
NUM_CHUNKS = ROWS_PER_WORKER // CHUNK
NBUF = 4

_MESH = plsc.VectorSubcoreMesh(
    core_axis_name="c", subcore_axis_name="s", num_cores=2, num_subcores=16
)


@functools.partial(
    pl.kernel,
    out_type=jax.ShapeDtypeStruct((N_ROWS, EMBED), jnp.float32),
    mesh=_MESH,
    compiler_params=pltpu.CompilerParams(use_tc_tiling_on_sc=False),
    scratch_types=[
        pltpu.VMEM((NUM_CHUNKS, CHUNK), jnp.int32),
        pltpu.VMEM((NUM_CHUNKS, CHUNK), jnp.int32),
    ]
    + [pltpu.VMEM((CHUNK, EMBED), jnp.float32) for _ in range(NBUF)]
    + [pltpu.SemaphoreType.DMA] * (3 * NBUF),
)
def _embed_sum(notes_hbm, times_hbm, tok_hbm, pos_hbm, out_hbm,
               idx_n, idx_t, *bufs_and_sems):
    bufs = bufs_and_sems[:NBUF]
    sem_a = bufs_and_sems[NBUF:2 * NBUF]
    sem_b = bufs_and_sems[2 * NBUF:3 * NBUF]
    sem_c = bufs_and_sems[3 * NBUF:4 * NBUF]

    w = lax.axis_index("s") * 2 + lax.axis_index("c")
    pltpu.sync_copy(notes_hbm.at[w], idx_n)
    pltpu.sync_copy(times_hbm.at[w], idx_t)
    out_base = w * ROWS_PER_WORKER

    def step(g):
        for b in range(NBUF):
            c = g + b
            b1 = (b - 1) % NBUF
            b2 = (b - 2) % NBUF

            @pl.when(jnp.logical_and(c >= NBUF, c - NBUF < NUM_CHUNKS))
            def _():
                pltpu.make_async_copy(
                    bufs[b], out_hbm.at[pl.ds(out_base, CHUNK)], sem_c[b]
                ).wait()

            @pl.when(c < NUM_CHUNKS)
            def _():
                pltpu.async_copy(tok_hbm.at[idx_n.at[c]], bufs[b], sem_a[b])

            c1 = c - 1
            @pl.when(jnp.logical_and(c1 >= 0, c1 < NUM_CHUNKS))
            def _():
                pltpu.make_async_copy(
                    tok_hbm.at[idx_n.at[c1]], bufs[b1], sem_a[b1]
                ).wait()
                pltpu.async_copy(
                    pos_hbm.at[idx_t.at[c1]], bufs[b1], sem_b[b1], add=True
                )

            c2 = c - 2
            @pl.when(jnp.logical_and(c2 >= 0, c2 < NUM_CHUNKS))
            def _():
                pltpu.make_async_copy(
                    pos_hbm.at[idx_t.at[c2]], bufs[b2], sem_b[b2]
                ).wait()
                pltpu.async_copy(
                    bufs[b2],
                    out_hbm.at[pl.ds(out_base + c2 * CHUNK, CHUNK)],
                    sem_c[b2],
                )

    pl.loop(0, NUM_CHUNKS + NBUF, step=NBUF)(step)


def kernel(x, token_emb, pos_emb):
    notes = x[:, 0, :].astype(jnp.int32).reshape(NUM_WORKERS, NUM_CHUNKS, CHUNK)
    times = x[:, 1, :].astype(jnp.int32).reshape(NUM_WORKERS, NUM_CHUNKS, CHUNK)
    out = _embed_sum(notes, times, token_emb, pos_emb)
    return out.reshape(BATCH, SEQ, EMBED)
